# baseline (speedup 1.0000x reference)
import functools

import jax
import jax.numpy as jnp
from jax import lax
from jax.experimental import pallas as pl
from jax.experimental.pallas import tpu as pltpu
from jax.experimental.pallas import tpu_sc as plsc

B, T, F = 16384, 200, 9
NW = 32


def _sc_body(x_t, o_st, o_kc,
             in_v0, in_v1, cvt_v0, cvt_v1, s_i0, s_i1, s_o0, s_o1):
    wid = lax.axis_index("s") * 2 + lax.axis_index("c")

    def conv_plane(plane, out_ref):
        in_bufs = (in_v0, in_v1)
        cvt_bufs = (cvt_v0, cvt_v1)
        in_sems = (s_i0, s_i1)
        out_sems = (s_o0, s_o1)

        def src_of(g):
            item = (wid & 15) * 25 + g
            return x_t.at[plane, pl.ds((item >> 4) * 8, 8),
                          pl.ds((item & 15) * 1024, 1024)]

        def dst_of(g):
            item = (wid & 15) * 25 + g
            return out_ref.at[pl.ds((item >> 4) * 8, 8), 0,
                              pl.ds((item & 15) * 1024, 1024)]

        pltpu.async_copy(src_of(0), in_bufs[0], in_sems[0])

        def step(g, carry):
            for b in range(2):
                @pl.when((g & 1) == b)
                def _(b=b):
                    pltpu.make_async_copy(src_of(g), in_bufs[b],
                                          in_sems[b]).wait()

                    @pl.when(g < 24)
                    def _():
                        pltpu.async_copy(src_of(g + 1), in_bufs[1 - b],
                                         in_sems[1 - b])

                    @pl.when(g >= 2)
                    def _():
                        pltpu.make_async_copy(cvt_bufs[b], dst_of(g - 2),
                                              out_sems[b]).wait()

                    def cvt(j, carry2):
                        for tr in range(8):
                            v = in_bufs[b][tr, pl.ds(j * 16, 16)]
                            cvt_bufs[b][tr, pl.ds(j * 16, 16)] = (
                                v.astype(jnp.int32))
                        return carry2

                    lax.fori_loop(0, 64, cvt, 0, unroll=4)
                    pltpu.async_copy(cvt_bufs[b], dst_of(g), out_sems[b])
            return carry

        lax.fori_loop(0, 25, step, 0)
        pltpu.make_async_copy(cvt_bufs[1], dst_of(23), out_sems[1]).wait()
        pltpu.make_async_copy(cvt_bufs[0], dst_of(24), out_sems[0]).wait()

    @pl.when(wid < 16)
    def _():
        conv_plane(0, o_st)

    @pl.when(wid >= 16)
    def _():
        conv_plane(5, o_kc)


def _tc_body(x_ref, ob_ref, kr_ref, sem_ob, sem_kr):
    for p in range(3):
        pltpu.make_async_copy(x_ref.at[6 + p], ob_ref.at[p], sem_ob).start()

    W = 16

    def kr_copy(t):
        return pltpu.make_async_copy(
            x_ref.at[pl.ds(1, 4), t, :], kr_ref.at[t], sem_kr)

    def fire(t, carry):
        kr_copy(t).start()

        @pl.when(t >= W)
        def _():
            kr_copy(t - W).wait()
        return carry

    lax.fori_loop(0, T, fire, 0)

    def drain(t, carry):
        kr_copy(t).wait()
        return carry

    lax.fori_loop(T - W, T, drain, 0)
    for p in range(3):
        pltpu.make_async_copy(x_ref.at[6 + p], ob_ref.at[p], sem_ob).wait()


@jax.jit
def _run(x_t):
    mesh = plsc.VectorSubcoreMesh(core_axis_name="c", subcore_axis_name="s")
    sc = pl.kernel(
        _sc_body,
        out_type=[
            jax.ShapeDtypeStruct((T, 1, B), jnp.int32),
            jax.ShapeDtypeStruct((T, 1, B), jnp.int32),
        ],
        mesh=mesh,
        compiler_params=pltpu.CompilerParams(
            needs_layout_passes=False, use_tc_tiling_on_sc=True
        ),
        scratch_types=[
            pltpu.VMEM((8, 1024), jnp.float32),
            pltpu.VMEM((8, 1024), jnp.float32),
            pltpu.VMEM((8, 1024), jnp.int32),
            pltpu.VMEM((8, 1024), jnp.int32),
            pltpu.SemaphoreType.DMA,
            pltpu.SemaphoreType.DMA,
            pltpu.SemaphoreType.DMA,
            pltpu.SemaphoreType.DMA,
        ],
    )
    st, kc = sc(x_t)

    ob, kr = pl.pallas_call(
        _tc_body,
        in_specs=[pl.BlockSpec(memory_space=pl.ANY)],
        out_specs=[pl.BlockSpec(memory_space=pl.ANY),
                   pl.BlockSpec(memory_space=pl.ANY)],
        out_shape=[
            jax.ShapeDtypeStruct((3, T, B), jnp.float32),
            jax.ShapeDtypeStruct((T, 4, B), jnp.float32),
        ],
        scratch_shapes=[pltpu.SemaphoreType.DMA, pltpu.SemaphoreType.DMA],
    )(x_t)
    return st, kr, kc, ob


def kernel(inputs):
    x_t = jnp.transpose(inputs, (2, 1, 0))
    st, kr, kc, ob = _run(x_t)
    return (
        jnp.transpose(st, (2, 0, 1)),
        jnp.transpose(kr, (2, 0, 1)),
        jnp.transpose(kc, (2, 0, 1)),
        jnp.transpose(ob, (2, 1, 0)),
    )


# speedup vs baseline: 21.0895x; 21.0895x over previous
import functools

import jax
import jax.numpy as jnp
from jax import lax
from jax.experimental import pallas as pl
from jax.experimental.pallas import tpu as pltpu
from jax.experimental.pallas import tpu_sc as plsc

B, T, F = 16384, 200, 9
NW = 32


def _staged_ring(lo, hi, src_of, dst_of, bufs, sems_i, sems_o):
    @pl.when(hi > lo)
    def _():
        pltpu.async_copy(src_of(lo), bufs[0], sems_i[0])

        def step(i, carry):
            for b in range(2):
                @pl.when(((i - lo) & 1) == b)
                def _(b=b):
                    pltpu.make_async_copy(src_of(i), bufs[b],
                                          sems_i[b]).wait()

                    @pl.when(i > lo)
                    def _():
                        pltpu.make_async_copy(bufs[1 - b], dst_of(i - 1),
                                              sems_o[1 - b]).wait()

                    @pl.when(i + 1 < hi)
                    def _():
                        pltpu.async_copy(src_of(i + 1), bufs[1 - b],
                                         sems_i[1 - b])

                    pltpu.async_copy(bufs[b], dst_of(i), sems_o[b])
            return carry

        lax.fori_loop(lo, hi, step, 0)
        for b in range(2):
            @pl.when(((hi - 1 - lo) & 1) == b)
            def _(b=b):
                pltpu.make_async_copy(bufs[b], dst_of(hi - 1),
                                      sems_o[b]).wait()


def _sc_body(x_t, o_st, o_kr, o_kc,
             kr0, kr1, in_v0, in_v1, cvt_v0, cvt_v1,
             s_i0, s_i1, s_o0, s_o1, c_i0, c_i1, c_o0, c_o1):
    wid = lax.axis_index("s") * 2 + lax.axis_index("c")

    def kr_src(i):
        return x_t.at[pl.ds(1, 4), i >> 2, pl.ds((i & 3) * 4096, 4096)]

    def kr_dst(i):
        return o_kr.at[i >> 2, :, pl.ds((i & 3) * 4096, 4096)]

    _staged_ring(wid * 25, (wid + 1) * 25, kr_src, kr_dst,
                 (kr0, kr1), (c_i0, c_i1), (c_o0, c_o1))

    def conv_plane(plane, out_ref):
        in_bufs = (in_v0, in_v1)
        cvt_bufs = (cvt_v0, cvt_v1)
        in_sems = (s_i0, s_i1)
        out_sems = (s_o0, s_o1)

        def src_of(g):
            item = (wid & 15) * 25 + g
            return x_t.at[plane, pl.ds((item >> 4) * 8, 8),
                          pl.ds((item & 15) * 1024, 1024)]

        def dst_of(g):
            item = (wid & 15) * 25 + g
            return out_ref.at[pl.ds((item >> 4) * 8, 8), 0,
                              pl.ds((item & 15) * 1024, 1024)]

        pltpu.async_copy(src_of(0), in_bufs[0], in_sems[0])

        def step(g, carry):
            for b in range(2):
                @pl.when((g & 1) == b)
                def _(b=b):
                    pltpu.make_async_copy(src_of(g), in_bufs[b],
                                          in_sems[b]).wait()

                    @pl.when(g < 24)
                    def _():
                        pltpu.async_copy(src_of(g + 1), in_bufs[1 - b],
                                         in_sems[1 - b])

                    @pl.when(g >= 2)
                    def _():
                        pltpu.make_async_copy(cvt_bufs[b], dst_of(g - 2),
                                              out_sems[b]).wait()

                    def cvt(j, carry2):
                        for tr in range(8):
                            v = in_bufs[b][tr, pl.ds(j * 16, 16)]
                            cvt_bufs[b][tr, pl.ds(j * 16, 16)] = (
                                v.astype(jnp.int32))
                        return carry2

                    lax.fori_loop(0, 64, cvt, 0, unroll=4)
                    pltpu.async_copy(cvt_bufs[b], dst_of(g), out_sems[b])
            return carry

        lax.fori_loop(0, 25, step, 0)
        pltpu.make_async_copy(cvt_bufs[1], dst_of(23), out_sems[1]).wait()
        pltpu.make_async_copy(cvt_bufs[0], dst_of(24), out_sems[0]).wait()

    @pl.when(wid < 16)
    def _():
        conv_plane(0, o_st)

    @pl.when(wid >= 16)
    def _():
        conv_plane(5, o_kc)


def _tc_ob_body(x_ref, ob_ref):
    ob_ref[...] = x_ref[...]


@jax.jit
def _run(x_t):
    mesh = plsc.VectorSubcoreMesh(core_axis_name="c", subcore_axis_name="s")
    sc = pl.kernel(
        _sc_body,
        out_type=[
            jax.ShapeDtypeStruct((T, 1, B), jnp.int32),
            jax.ShapeDtypeStruct((T, 4, B), jnp.float32),
            jax.ShapeDtypeStruct((T, 1, B), jnp.int32),
        ],
        mesh=mesh,
        compiler_params=pltpu.CompilerParams(
            needs_layout_passes=False, use_tc_tiling_on_sc=True
        ),
        scratch_types=[
            pltpu.VMEM((4, 4096), jnp.float32),
            pltpu.VMEM((4, 4096), jnp.float32),
            pltpu.VMEM((8, 1024), jnp.float32),
            pltpu.VMEM((8, 1024), jnp.float32),
            pltpu.VMEM((8, 1024), jnp.int32),
            pltpu.VMEM((8, 1024), jnp.int32),
            pltpu.SemaphoreType.DMA,
            pltpu.SemaphoreType.DMA,
            pltpu.SemaphoreType.DMA,
            pltpu.SemaphoreType.DMA,
            pltpu.SemaphoreType.DMA,
            pltpu.SemaphoreType.DMA,
            pltpu.SemaphoreType.DMA,
            pltpu.SemaphoreType.DMA,
        ],
    )
    st, kr, kc = sc(x_t)

    ob = pl.pallas_call(
        _tc_ob_body,
        grid=(75,),
        in_specs=[pl.BlockSpec((1, 8, B), lambda i: (6 + i // 25, i % 25, 0))],
        out_specs=pl.BlockSpec((1, 8, B), lambda i: (i // 25, i % 25, 0)),
        out_shape=jax.ShapeDtypeStruct((3, T, B), jnp.float32),
    )(x_t)
    return st, kr, kc, ob


def kernel(inputs):
    x_t = jnp.transpose(inputs, (2, 1, 0))
    st, kr, kc, ob = _run(x_t)
    return (
        jnp.transpose(st, (2, 0, 1)),
        jnp.transpose(kr, (2, 0, 1)),
        jnp.transpose(kc, (2, 0, 1)),
        jnp.transpose(ob, (2, 1, 0)),
    )


# speedup vs baseline: 22.4608x; 1.0650x over previous
import functools

import jax
import jax.numpy as jnp
from jax import lax
from jax.experimental import pallas as pl
from jax.experimental.pallas import tpu as pltpu
from jax.experimental.pallas import tpu_sc as plsc

B, T, F = 16384, 200, 9
NW = 32


def _staged_ring(lo, hi, src_of, dst_of, bufs, sems_i, sems_o):
    @pl.when(hi > lo)
    def _():
        pltpu.async_copy(src_of(lo), bufs[0], sems_i[0])

        def step(i, carry):
            for b in range(2):
                @pl.when(((i - lo) & 1) == b)
                def _(b=b):
                    pltpu.make_async_copy(src_of(i), bufs[b],
                                          sems_i[b]).wait()

                    @pl.when(i > lo)
                    def _():
                        pltpu.make_async_copy(bufs[1 - b], dst_of(i - 1),
                                              sems_o[1 - b]).wait()

                    @pl.when(i + 1 < hi)
                    def _():
                        pltpu.async_copy(src_of(i + 1), bufs[1 - b],
                                         sems_i[1 - b])

                    pltpu.async_copy(bufs[b], dst_of(i), sems_o[b])
            return carry

        lax.fori_loop(lo, hi, step, 0)
        for b in range(2):
            @pl.when(((hi - 1 - lo) & 1) == b)
            def _(b=b):
                pltpu.make_async_copy(bufs[b], dst_of(hi - 1),
                                      sems_o[b]).wait()


def _sc_body(x_t, o_st, o_kr, o_kc,
             kr0, kr1, in_v0, in_v1, cvt_v0, cvt_v1,
             s_i0, s_i1, s_o0, s_o1, c_i0, c_i1, c_o0, c_o1):
    wid = lax.axis_index("s") * 2 + lax.axis_index("c")

    def kr_src(i):
        return x_t.at[pl.ds(1, 4), i >> 1, pl.ds((i & 1) * 8192, 8192)]

    def kr_dst(i):
        return o_kr.at[i >> 1, :, pl.ds((i & 1) * 8192, 8192)]

    _staged_ring((wid * 25) >> 1, (((wid + 1) * 25) >> 1), kr_src, kr_dst,
                 (kr0, kr1), (c_i0, c_i1), (c_o0, c_o1))

    def conv_plane(plane, out_ref):
        in_bufs = (in_v0, in_v1)
        cvt_bufs = (cvt_v0, cvt_v1)
        in_sems = (s_i0, s_i1)
        out_sems = (s_o0, s_o1)

        def src_of(g):
            item = (wid & 15) * 25 + g
            return x_t.at[plane, pl.ds((item >> 4) * 8, 8),
                          pl.ds((item & 15) * 1024, 1024)]

        def dst_of(g):
            item = (wid & 15) * 25 + g
            return out_ref.at[pl.ds((item >> 4) * 8, 8), 0,
                              pl.ds((item & 15) * 1024, 1024)]

        pltpu.async_copy(src_of(0), in_bufs[0], in_sems[0])

        def step(g, carry):
            for b in range(2):
                @pl.when((g & 1) == b)
                def _(b=b):
                    pltpu.make_async_copy(src_of(g), in_bufs[b],
                                          in_sems[b]).wait()

                    @pl.when(g < 24)
                    def _():
                        pltpu.async_copy(src_of(g + 1), in_bufs[1 - b],
                                         in_sems[1 - b])

                    @pl.when(g >= 2)
                    def _():
                        pltpu.make_async_copy(cvt_bufs[b], dst_of(g - 2),
                                              out_sems[b]).wait()

                    def cvt(j, carry2):
                        for tr in range(8):
                            v = in_bufs[b][tr, pl.ds(j * 16, 16)]
                            cvt_bufs[b][tr, pl.ds(j * 16, 16)] = (
                                v.astype(jnp.int32))
                        return carry2

                    lax.fori_loop(0, 64, cvt, 0, unroll=4)
                    pltpu.async_copy(cvt_bufs[b], dst_of(g), out_sems[b])
            return carry

        lax.fori_loop(0, 25, step, 0)
        pltpu.make_async_copy(cvt_bufs[1], dst_of(23), out_sems[1]).wait()
        pltpu.make_async_copy(cvt_bufs[0], dst_of(24), out_sems[0]).wait()

    @pl.when(wid < 16)
    def _():
        conv_plane(0, o_st)

    @pl.when(wid >= 16)
    def _():
        conv_plane(5, o_kc)


def _tc_ob_body(x_ref, ob_ref):
    ob_ref[...] = x_ref[...]


@jax.jit
def _run(x_t):
    mesh = plsc.VectorSubcoreMesh(core_axis_name="c", subcore_axis_name="s")
    sc = pl.kernel(
        _sc_body,
        out_type=[
            jax.ShapeDtypeStruct((T, 1, B), jnp.int32),
            jax.ShapeDtypeStruct((T, 4, B), jnp.float32),
            jax.ShapeDtypeStruct((T, 1, B), jnp.int32),
        ],
        mesh=mesh,
        compiler_params=pltpu.CompilerParams(
            needs_layout_passes=False, use_tc_tiling_on_sc=True
        ),
        scratch_types=[
            pltpu.VMEM((4, 8192), jnp.float32),
            pltpu.VMEM((4, 8192), jnp.float32),
            pltpu.VMEM((8, 1024), jnp.float32),
            pltpu.VMEM((8, 1024), jnp.float32),
            pltpu.VMEM((8, 1024), jnp.int32),
            pltpu.VMEM((8, 1024), jnp.int32),
            pltpu.SemaphoreType.DMA,
            pltpu.SemaphoreType.DMA,
            pltpu.SemaphoreType.DMA,
            pltpu.SemaphoreType.DMA,
            pltpu.SemaphoreType.DMA,
            pltpu.SemaphoreType.DMA,
            pltpu.SemaphoreType.DMA,
            pltpu.SemaphoreType.DMA,
        ],
    )
    st, kr, kc = sc(x_t)

    ob = pl.pallas_call(
        _tc_ob_body,
        grid=(75,),
        in_specs=[pl.BlockSpec((1, 8, B), lambda i: (6 + i // 25, i % 25, 0))],
        out_specs=pl.BlockSpec((1, 8, B), lambda i: (i // 25, i % 25, 0)),
        out_shape=jax.ShapeDtypeStruct((3, T, B), jnp.float32),
    )(x_t)
    return st, kr, kc, ob


def kernel(inputs):
    x_t = jnp.transpose(inputs, (2, 1, 0))
    st, kr, kc, ob = _run(x_t)
    return (
        jnp.transpose(st, (2, 0, 1)),
        jnp.transpose(kr, (2, 0, 1)),
        jnp.transpose(kc, (2, 0, 1)),
        jnp.transpose(ob, (2, 1, 0)),
    )
